# baseline (speedup 1.0000x reference)
import functools

import jax
import jax.numpy as jnp
from jax import lax
from jax.experimental import pallas as pl
from jax.experimental.pallas import tpu as pltpu
from jax.experimental.pallas import tpu_sc as plsc

N_ROWS = 6_400_000
_INFO = plsc.get_sparse_core_info()
_NC = _INFO.num_cores
_NS = _INFO.num_subcores
_NW = _NC * _NS
ROWS_PER_W = N_ROWS // _NW
CHUNK = 2_000
N_CHUNKS = ROWS_PER_W // CHUNK
GROUPS = CHUNK // 16

_mesh = plsc.VectorSubcoreMesh(core_axis_name="c", subcore_axis_name="s")


@functools.partial(
    pl.kernel,
    mesh=_mesh,
    out_type=jax.ShapeDtypeStruct((N_ROWS * 2,), jnp.float32),
    scratch_types=[
        pltpu.VMEM((CHUNK * 5,), jnp.float32),
        pltpu.VMEM((CHUNK * 2,), jnp.float32),
    ],
    compiler_params=pltpu.CompilerParams(needs_layout_passes=False),
)
def _segment_prod_sc(x_hbm, out_hbm, in_v, out_v):
    wid = lax.axis_index("s") * _NC + lax.axis_index("c")

    lanes = lax.iota(jnp.int32, 16)
    in_lanes = lanes * 5
    out_lanes = lanes * 2

    def chunk_body(it, _):
        row0 = wid * ROWS_PER_W + it * CHUNK
        pltpu.sync_copy(x_hbm.at[pl.ds(row0 * 5, CHUNK * 5)], in_v)

        def group_body(g, _):
            idx = g * 80 + in_lanes
            a = plsc.load_gather(in_v, [idx])
            b = plsc.load_gather(in_v, [idx + 1])
            c = plsc.load_gather(in_v, [idx + 2])
            d = plsc.load_gather(in_v, [idx + 3])
            e = plsc.load_gather(in_v, [idx + 4])
            oidx = g * 32 + out_lanes
            plsc.store_scatter(out_v, [oidx], a * b * c)
            plsc.store_scatter(out_v, [oidx + 1], d * e)
            return 0

        lax.fori_loop(0, GROUPS, group_body, 0)
        pltpu.sync_copy(out_v, out_hbm.at[pl.ds(row0 * 2, CHUNK * 2)])
        return 0

    lax.fori_loop(0, N_CHUNKS, chunk_body, 0)


def kernel(x):
    flat = _segment_prod_sc(x.reshape(-1))
    return flat.reshape(N_ROWS, 2)


# speedup vs baseline: 1.0205x; 1.0205x over previous
import functools

import jax
import jax.numpy as jnp
from jax import lax
from jax.experimental import pallas as pl
from jax.experimental.pallas import tpu as pltpu
from jax.experimental.pallas import tpu_sc as plsc

N_ROWS = 6_400_000
_INFO = plsc.get_sparse_core_info()
_NC = _INFO.num_cores
_NS = _INFO.num_subcores
_NW = _NC * _NS
ROWS_PER_W = N_ROWS // _NW
CHUNK = 4_000
N_CHUNKS = ROWS_PER_W // CHUNK
GROUPS = CHUNK // 16
NBUF = 2

_mesh = plsc.VectorSubcoreMesh(core_axis_name="c", subcore_axis_name="s")


@functools.partial(
    pl.kernel,
    mesh=_mesh,
    out_type=jax.ShapeDtypeStruct((N_ROWS * 2,), jnp.float32),
    scratch_types=[
        pltpu.VMEM((CHUNK * 5,), jnp.float32),
        pltpu.VMEM((CHUNK * 5,), jnp.float32),
        pltpu.VMEM((CHUNK * 2,), jnp.float32),
        pltpu.VMEM((CHUNK * 2,), jnp.float32),
        pltpu.SemaphoreType.DMA,
        pltpu.SemaphoreType.DMA,
        pltpu.SemaphoreType.DMA,
        pltpu.SemaphoreType.DMA,
    ],
    compiler_params=pltpu.CompilerParams(needs_layout_passes=False),
)
def _segment_prod_sc(x_hbm, out_hbm, in0, in1, out0, out1, si0, si1, so0, so1):
    wid = lax.axis_index("s") * _NC + lax.axis_index("c")
    in_base = wid * (ROWS_PER_W * 5)
    out_base = wid * (ROWS_PER_W * 2)

    ins = (in0, in1)
    outs = (out0, out1)
    sis = (si0, si1)
    sos = (so0, so1)

    lanes = lax.iota(jnp.int32, 16)
    in_lanes = lanes * 5
    out_lanes = lanes * 2

    for b in range(NBUF):
        pltpu.async_copy(
            x_hbm.at[pl.ds(in_base + b * (CHUNK * 5), CHUNK * 5)],
            ins[b], sis[b])

    def outer(j, _):
        for b in range(NBUF):
            it = j * NBUF + b
            iv, ov, si, so = ins[b], outs[b], sis[b], sos[b]

            pltpu.make_async_copy(
                x_hbm.at[pl.ds(0, CHUNK * 5)], iv, si).wait()
            @pl.when(it >= NBUF)
            def _():
                pltpu.make_async_copy(
                    x_hbm.at[pl.ds(0, CHUNK * 2)], ov, so).wait()

            @plsc.parallel_loop(0, GROUPS, unroll=10)
            def _(g):
                idx = g * 80 + in_lanes
                a = plsc.load_gather(iv, [idx])
                bb = plsc.load_gather(iv, [idx + 1])
                c = plsc.load_gather(iv, [idx + 2])
                d = plsc.load_gather(iv, [idx + 3])
                e = plsc.load_gather(iv, [idx + 4])
                oidx = g * 32 + out_lanes
                plsc.store_scatter(ov, [oidx], a * bb * c)
                plsc.store_scatter(ov, [oidx + 1], d * e)

            pltpu.async_copy(
                ov, out_hbm.at[pl.ds(out_base + it * (CHUNK * 2),
                                     CHUNK * 2)], so)

            @pl.when(it + NBUF < N_CHUNKS)
            def _():
                pltpu.async_copy(
                    x_hbm.at[pl.ds(in_base + (it + NBUF) * (CHUNK * 5),
                                   CHUNK * 5)],
                    iv, si)
        return 0

    lax.fori_loop(0, N_CHUNKS // NBUF, outer, 0)

    for b in range(NBUF):
        pltpu.make_async_copy(
            x_hbm.at[pl.ds(0, CHUNK * 2)], outs[b], sos[b]).wait()


def kernel(x):
    flat = _segment_prod_sc(x.reshape(-1))
    return flat.reshape(N_ROWS, 2)
